# baseline (speedup 1.0000x reference)
import functools

import jax
import jax.numpy as jnp
from jax import lax
from jax.experimental import pallas as pl
from jax.experimental.pallas import tpu as pltpu
from jax.experimental.pallas import tpu_sc as plsc

_NC = 2
_NS = 16
_NW = _NC * _NS


@functools.lru_cache(maxsize=None)
def _make_gather(R: int, T: int, D: int, NBUF: int):
    SL = NBUF // 2
    rpw = R // _NW
    ngrp = rpw // NBUF
    assert R % _NW == 0 and rpw % NBUF == 0 and ngrp >= 2
    mesh = plsc.VectorSubcoreMesh(core_axis_name="c", subcore_axis_name="s")

    @functools.partial(
        pl.kernel,
        mesh=mesh,
        out_type=jax.ShapeDtypeStruct((R, T, D), jnp.float32),
        scratch_types=[
            pltpu.VMEM((rpw, T), jnp.int32),
            pltpu.VMEM((NBUF, T, D), jnp.float32),
        ]
        + [pltpu.SemaphoreType.DMA] * (2 * NBUF),
        compiler_params=pltpu.CompilerParams(use_tc_tiling_on_sc=False),
    )
    def emb_gather(idx_hbm, table_hbm, out_hbm, idx_v, rows_v, *sems):
        gsem = sems[:NBUF]
        ssem = sems[NBUF:]
        wid = lax.axis_index("s") * _NC + lax.axis_index("c")
        base = wid * rpw

        pltpu.sync_copy(idx_hbm.at[pl.ds(base, rpw)], idx_v)

        def start_gather(r, b):
            pltpu.async_copy(table_hbm.at[idx_v.at[r]], rows_v.at[b], gsem[b])

        def start_store(r, b):
            pltpu.async_copy(rows_v.at[b], out_hbm.at[base + r], ssem[b])

        def wait_g(b):
            pltpu.make_async_copy(rows_v.at[b], out_hbm.at[0], gsem[b]).wait()

        def wait_s(b):
            pltpu.make_async_copy(rows_v.at[b], out_hbm.at[0], ssem[b]).wait()

        for b in range(NBUF):
            start_gather(b, b)
            if b >= SL:
                wait_g(b - SL)
                start_store(b - SL, b - SL)

        def group(g, carry):
            r0 = g * NBUF
            for b in range(NBUF):
                wait_s(b)
                start_gather(r0 + b, b)
                b2 = (b - SL) % NBUF
                wait_g(b2)
                start_store(r0 + b - SL, b2)
            return carry

        lax.fori_loop(1, ngrp, group, 0)

        for k in range(SL):
            r = rpw - SL + k
            b = r % NBUF
            wait_g(b)
            start_store(r, b)
        for b in range(NBUF):
            wait_s(b)

    return emb_gather


def kernel(x, emb_loc):
    R, T = x.shape
    D = emb_loc.shape[1]
    return _make_gather(R, T, D, 4)(x.astype(jnp.int32), emb_loc)


# speedup vs baseline: 1.2252x; 1.2252x over previous
import functools

import jax
import jax.numpy as jnp
from jax import lax
from jax.experimental import pallas as pl
from jax.experimental.pallas import tpu as pltpu
from jax.experimental.pallas import tpu_sc as plsc

_NC = 2
_NS = 16
_NW = _NC * _NS


@functools.lru_cache(maxsize=None)
def _make_gather(R: int, T: int, D: int, DP: int, NBUF: int, SL: int):
    rpw = R // _NW
    ngrp = rpw // NBUF
    assert R % _NW == 0 and rpw % NBUF == 0 and ngrp >= 2 and 1 <= SL < NBUF
    mesh = plsc.VectorSubcoreMesh(core_axis_name="c", subcore_axis_name="s")

    @functools.partial(
        pl.kernel,
        mesh=mesh,
        out_type=jax.ShapeDtypeStruct((R * T, D), jnp.float32),
        scratch_types=[
            pltpu.VMEM((rpw * T,), jnp.int32),
            pltpu.VMEM((NBUF, T, DP), jnp.float32),
            pltpu.VMEM((NBUF, T, D), jnp.float32),
        ]
        + [pltpu.SemaphoreType.DMA] * (3 * NBUF),
    )
    def emb_gather(idx_hbm, table_hbm, out_hbm, idx_v, rows_g, rows_s, *sems):
        gsem = sems[:NBUF]
        psem = sems[NBUF:2 * NBUF]
        ssem = sems[2 * NBUF:]
        wid = lax.axis_index("s") * _NC + lax.axis_index("c")
        base = wid * rpw

        pltpu.sync_copy(idx_hbm.at[pl.ds(base * T, rpw * T)], idx_v)

        def start_gather(r, b):
            pltpu.async_copy(
                table_hbm.at[idx_v.at[pl.ds(r * T, T)]], rows_g.at[b], gsem[b]
            )

        def repack(b):
            def rows8(i, carry):
                t0 = i * 8
                for dt in range(8):
                    for k in range(D // 16):
                        rows_s[b, t0 + dt, pl.ds(k * 16, 16)] = (
                            rows_g[b, t0 + dt, pl.ds(k * 16, 16)]
                        )
                return carry
            lax.fori_loop(0, T // 8, rows8, 0)

        def start_store(r, b):
            pltpu.async_copy(
                rows_s.at[b], out_hbm.at[pl.ds((base + r) * T, T)], ssem[b]
            )

        def wait_g(b):
            pltpu.make_async_copy(rows_g.at[b], out_hbm.at[pl.ds(0, T)],
                                  gsem[b]).wait()

        def wait_s(b):
            pltpu.make_async_copy(rows_s.at[b], out_hbm.at[pl.ds(0, T)],
                                  ssem[b]).wait()

        for b in range(NBUF):
            start_gather(b, b)
            if b >= SL:
                wait_g(b - SL)
                repack(b - SL)
                start_store(b - SL, b - SL)

        def group(g, carry):
            r0 = g * NBUF
            for b in range(NBUF):
                wait_s(b)
                start_gather(r0 + b, b)
                b2 = (b - SL) % NBUF
                wait_g(b2)
                repack(b2)
                start_store(r0 + b - SL, b2)
            return carry

        lax.fori_loop(1, ngrp, group, 0)

        for k in range(SL):
            r = rpw - SL + k
            b = r % NBUF
            wait_g(b)
            repack(b)
            start_store(r, b)
        for b in range(NBUF):
            wait_s(b)

    return emb_gather


def kernel(x, emb_loc):
    R, T = x.shape
    V, D = emb_loc.shape
    DP = 128
    VP = (V + 7) // 8 * 8
    emb_p = jnp.pad(emb_loc, ((0, VP - V), (0, DP - D)))
    xf = x.reshape(-1).astype(jnp.int32)
    out = _make_gather(R, T, D, DP, 2, 1)(xf, emb_p)
    return out.reshape(R, T, D)
